# baseline (speedup 1.0000x reference)
import functools

import jax
import jax.numpy as jnp
from jax import lax
from jax.experimental import pallas as pl
from jax.experimental.pallas import tpu as pltpu
from jax.experimental.pallas import tpu_sc as plsc


def _sc_scores(W1, W2, pos_c, pos_n, neg_n):
    B = pos_c.shape[0]
    D = W1.shape[1]
    info = plsc.get_sparse_core_info()
    NC, NS, L = info.num_cores, info.num_subcores, info.num_lanes
    NW = NC * NS
    b_per_w = B // NW
    mesh = plsc.VectorSubcoreMesh(core_axis_name="c", subcore_axis_name="s")

    @functools.partial(
        pl.kernel,
        out_type=(
            jax.ShapeDtypeStruct((B,), jnp.float32),
            jax.ShapeDtypeStruct((B,), jnp.float32),
        ),
        mesh=mesh,
        compiler_params=pltpu.CompilerParams(use_tc_tiling_on_sc=False),
        scratch_types=[
            pltpu.VMEM((b_per_w,), jnp.int32),
            pltpu.VMEM((b_per_w,), jnp.int32),
            pltpu.VMEM((b_per_w,), jnp.int32),
            pltpu.VMEM((b_per_w, D), jnp.float32),
            pltpu.VMEM((b_per_w, D), jnp.float32),
            pltpu.VMEM((b_per_w, D), jnp.float32),
            pltpu.VMEM((b_per_w,), jnp.float32),
            pltpu.VMEM((b_per_w,), jnp.float32),
            pltpu.SemaphoreType.DMA,
            pltpu.SemaphoreType.DMA,
            pltpu.SemaphoreType.DMA,
        ],
    )
    def sc_kernel(w1_hbm, w2_hbm, pc_hbm, pn_hbm, nn_hbm, pos_out, neg_out,
                  iu_v, iv_v, in_v, u_v, v_v, n_v, ps_v, ns_v, s0, s1, s2):
        wid = lax.axis_index("s") * NC + lax.axis_index("c")
        base = wid * b_per_w
        pltpu.sync_copy(pc_hbm.at[pl.ds(base, b_per_w)], iu_v)
        pltpu.sync_copy(pn_hbm.at[pl.ds(base, b_per_w)], iv_v)
        pltpu.sync_copy(nn_hbm.at[pl.ds(base, b_per_w)], in_v)
        cu = pltpu.async_copy(w1_hbm.at[iu_v], u_v, s0)
        cv = pltpu.async_copy(w2_hbm.at[iv_v], v_v, s1)
        cn = pltpu.async_copy(w2_hbm.at[in_v], n_v, s2)
        cu.wait()
        cv.wait()
        cn.wait()

        lanes = lax.iota(jnp.int32, L)
        dn = lax.GatherDimensionNumbers(
            offset_dims=(), collapsed_slice_dims=(0,), start_index_map=(0,))

        def perm(x, idx):
            return lax.gather(x, idx[:, None], dn, (1,),
                              mode=lax.GatherScatterMode.PROMISE_IN_BOUNDS)

        def hsum16(vecs):
            for s in range(4):
                m = 1 << s
                mask = (lanes & m) == 0
                pidx = lanes ^ m
                nxt = []
                for k in range(0, len(vecs), 2):
                    a, b = vecs[k], vecs[k + 1]
                    nxt.append(jnp.where(mask, a, perm(b, pidx))
                               + jnp.where(mask, perm(a, pidx), b))
                vecs = nxt
            return vecs[0]

        def group(g, _):
            pps, nns = [], []
            for j in range(L):
                b = g * L + j
                pacc = jnp.zeros((L,), jnp.float32)
                nacc = jnp.zeros((L,), jnp.float32)
                for k in range(D // L):
                    u = u_v[b, pl.ds(k * L, L)]
                    pacc = pacc + u * v_v[b, pl.ds(k * L, L)]
                    nacc = nacc + u * n_v[b, pl.ds(k * L, L)]
                pps.append(pacc)
                nns.append(nacc)
            ps_v[pl.ds(g * L, L)] = hsum16(pps)
            ns_v[pl.ds(g * L, L)] = hsum16(nns)
            return ()

        lax.fori_loop(0, b_per_w // L, group, ())
        pltpu.sync_copy(ps_v, pos_out.at[pl.ds(base, b_per_w)])
        pltpu.sync_copy(ns_v, neg_out.at[pl.ds(base, b_per_w)])

    return sc_kernel(W1, W2, pos_c, pos_n, neg_n)


def _tc_loss(pos_score, neg_score):
    B = pos_score.shape[0]
    ps2 = pos_score.reshape(B // 128, 128)
    ns2 = neg_score.reshape(B // 128, 128)

    def tc_kernel(ps_ref, ns_ref, o_ref):
        loss = jax.nn.log_sigmoid(ps_ref[...]) + jax.nn.log_sigmoid(-ns_ref[...])
        o_ref[0, 0] = -jnp.sum(loss) / B

    out = pl.pallas_call(
        tc_kernel,
        out_shape=jax.ShapeDtypeStruct((1, 1), jnp.float32),
        out_specs=pl.BlockSpec(memory_space=pltpu.SMEM),
    )(ps2, ns2)
    return out[0, 0]


def kernel(W1, W2, pos_c, pos_n, neg_n, batch_size):
    pos_score, neg_score = _sc_scores(
        W1, W2,
        pos_c.astype(jnp.int32), pos_n.astype(jnp.int32), neg_n.astype(jnp.int32),
    )
    return _tc_loss(pos_score, neg_score)
